# baseline (speedup 1.0000x reference)
import functools

import jax
import jax.numpy as jnp
from jax import lax
from jax.experimental import pallas as pl
from jax.experimental.pallas import tpu as pltpu
from jax.experimental.pallas import tpu_sc as plsc

D_MODEL = 768
LANES = 16
VECS_PER_ROW = D_MODEL // LANES
NUM_WORKERS = 32
CHUNK = 16
NBUF = 8
LOOKAHEAD = NBUF - 2


@functools.partial(jax.jit, static_argnames=("batch", "seq"))
def _emb_lookup_add(idx, token_table, pos_table, batch, seq):
    n = batch * seq
    pos_per_w = seq // NUM_WORKERS
    per_w = pos_per_w * batch
    n_parts = pos_per_w // CHUNK
    n_chunks = n_parts * batch
    mesh = plsc.VectorSubcoreMesh(core_axis_name="c", subcore_axis_name="s")

    @functools.partial(
        pl.kernel,
        mesh=mesh,
        out_type=jax.ShapeDtypeStruct((n, D_MODEL), jnp.float32),
        scratch_types=[
            pltpu.VMEM((per_w,), jnp.int32),
            pltpu.VMEM((CHUNK, D_MODEL), jnp.float32),
        ]
        + [pltpu.VMEM((CHUNK, D_MODEL), jnp.float32) for _ in range(NBUF)]
        + [pltpu.SemaphoreType.DMA for _ in range(2 * NBUF + 2)],
    )
    def k(idx_hbm, tok_hbm, pos_hbm, out_hbm, idx_v, posb, *bufs_sems):
        tokb = bufs_sems[:NBUF]
        gsem = bufs_sems[NBUF : 2 * NBUF]
        ssem = bufs_sems[2 * NBUF : 3 * NBUF]
        isem = bufs_sems[3 * NBUF]
        psem = bufs_sems[3 * NBUF + 1]

        wid = lax.axis_index("s") * 2 + lax.axis_index("c")
        pstart = wid * pos_per_w

        idx_cps = [
            pltpu.async_copy(
                idx_hbm.at[b, pl.ds(pstart, pos_per_w)],
                idx_v.at[pl.ds(b * pos_per_w, pos_per_w)],
                isem,
            )
            for b in range(batch)
        ]
        for cp in idx_cps:
            cp.wait()

        def chunk_gb(ck):
            return divmod(ck, batch)

        def start_gather(ck):
            g, b = chunk_gb(ck)
            return pltpu.async_copy(
                tok_hbm.at[idx_v.at[pl.ds(b * pos_per_w + g * CHUNK, CHUNK)]],
                tokb[ck % NBUF],
                gsem[ck % NBUF],
            )

        def out_row(ck):
            g, b = chunk_gb(ck)
            return b * seq + pstart + g * CHUNK

        gather_cps = {c: start_gather(c) for c in range(LOOKAHEAD)}
        store_cps = {}
        pos_cp = pltpu.async_copy(pos_hbm.at[pl.ds(pstart, CHUNK)], posb, psem)
        pos_cp.wait()

        for ck in range(n_chunks):
            p = ck % NBUF
            g, b = chunk_gb(ck)
            if b == 0 and g > 0:
                pltpu.sync_copy(pos_hbm.at[pl.ds(pstart + g * CHUNK, CHUNK)], posb)
            gather_cps[ck].wait()
            nk = ck + LOOKAHEAD
            if nk < n_chunks:
                if nk - NBUF >= 0:
                    store_cps[nk - NBUF].wait()
                gather_cps[nk] = start_gather(nk)

            buf = tokb[p]

            def row_body(r, _, buf=buf):
                for j in range(VECS_PER_ROW):
                    sl = pl.ds(j * LANES, LANES)
                    buf[r, sl] = buf[r, sl] + posb[r, sl]
                return 0

            lax.fori_loop(0, CHUNK, row_body, 0)

            store_cps[ck] = pltpu.async_copy(
                buf, out_hbm.at[pl.ds(out_row(ck), CHUNK)], ssem[p]
            )

        for ck in range(n_chunks - NBUF, n_chunks):
            store_cps[ck].wait()

    return k(idx, token_table, pos_table)


def kernel(inputs, token_table, pos_table):
    batch, seq = inputs.shape
    out = _emb_lookup_add(
        inputs.astype(jnp.int32), token_table, pos_table, batch, seq
    )
    return out.reshape(batch, seq, token_table.shape[1])


# speedup vs baseline: 1.0427x; 1.0427x over previous
import functools

import jax
import jax.numpy as jnp
from jax import lax
from jax.experimental import pallas as pl
from jax.experimental.pallas import tpu as pltpu
from jax.experimental.pallas import tpu_sc as plsc

D_MODEL = 768
LANES = 16
VECS_PER_ROW = D_MODEL // LANES
NUM_WORKERS = 32
CHUNK = 32
NBUF = 4


@functools.partial(jax.jit, static_argnames=("batch", "seq"))
def _emb_lookup_add(idx, token_table, pos_table, batch, seq):
    n = batch * seq
    pos_per_w = seq // NUM_WORKERS
    per_w = pos_per_w * batch
    n_halves = pos_per_w // CHUNK
    n_chunks = n_halves * batch
    mesh = plsc.VectorSubcoreMesh(core_axis_name="c", subcore_axis_name="s")

    @functools.partial(
        pl.kernel,
        mesh=mesh,
        out_type=jax.ShapeDtypeStruct((n, D_MODEL), jnp.float32),
        scratch_types=[
            pltpu.VMEM((per_w,), jnp.int32),
            pltpu.VMEM((CHUNK, D_MODEL), jnp.float32),
        ]
        + [pltpu.VMEM((CHUNK, D_MODEL), jnp.float32) for _ in range(NBUF)]
        + [pltpu.SemaphoreType.DMA for _ in range(2 * NBUF + 2)],
    )
    def k(idx_hbm, tok_hbm, pos_hbm, out_hbm, idx_v, posb, *bufs_sems):
        tokb = bufs_sems[:NBUF]
        gsem = bufs_sems[NBUF : 2 * NBUF]
        ssem = bufs_sems[2 * NBUF : 3 * NBUF]
        isem = bufs_sems[3 * NBUF]
        psem = bufs_sems[3 * NBUF + 1]

        wid = lax.axis_index("s") * 2 + lax.axis_index("c")
        pstart = wid * pos_per_w

        idx_cps = [
            pltpu.async_copy(
                idx_hbm.at[b, pl.ds(pstart, pos_per_w)],
                idx_v.at[pl.ds(b * pos_per_w, pos_per_w)],
                isem,
            )
            for b in range(batch)
        ]
        for cp in idx_cps:
            cp.wait()

        def chunk_gb(ck):
            return divmod(ck, batch)

        def start_gather(ck):
            g, b = chunk_gb(ck)
            return pltpu.async_copy(
                tok_hbm.at[idx_v.at[pl.ds(b * pos_per_w + g * CHUNK, CHUNK)]],
                tokb[ck % NBUF],
                gsem[ck % NBUF],
            )

        def out_row(ck):
            g, b = chunk_gb(ck)
            return b * seq + pstart + g * CHUNK

        gather_cps = {c: start_gather(c) for c in range(NBUF - 1)}
        store_cps = {}
        pos_cp = pltpu.async_copy(pos_hbm.at[pl.ds(pstart, CHUNK)], posb, psem)
        pos_cp.wait()

        for ck in range(n_chunks):
            p = ck % NBUF
            g, b = chunk_gb(ck)
            if b == 0 and g > 0:
                pltpu.sync_copy(pos_hbm.at[pl.ds(pstart + g * CHUNK, CHUNK)], posb)
            gather_cps[ck].wait()
            buf = tokb[p]

            def row_body(r, _, buf=buf):
                for j in range(VECS_PER_ROW):
                    sl = pl.ds(j * LANES, LANES)
                    buf[r, sl] = buf[r, sl] + posb[r, sl]
                return 0

            lax.fori_loop(0, CHUNK // 2, row_body, 0)
            nk = ck + NBUF - 1
            if nk < n_chunks:
                if nk - NBUF >= 0:
                    store_cps[nk - NBUF].wait()
                gather_cps[nk] = start_gather(nk)
            lax.fori_loop(CHUNK // 2, CHUNK, row_body, 0)
            store_cps[ck] = pltpu.async_copy(
                buf, out_hbm.at[pl.ds(out_row(ck), CHUNK)], ssem[p]
            )

        for ck in range(n_chunks - NBUF, n_chunks):
            store_cps[ck].wait()

    return k(idx, token_table, pos_table)


def kernel(inputs, token_table, pos_table):
    batch, seq = inputs.shape
    out = _emb_lookup_add(
        inputs.astype(jnp.int32), token_table, pos_table, batch, seq
    )
    return out.reshape(batch, seq, token_table.shape[1])


# speedup vs baseline: 1.0936x; 1.0489x over previous
import functools

import jax
import jax.numpy as jnp
from jax import lax
from jax.experimental import pallas as pl
from jax.experimental.pallas import tpu as pltpu
from jax.experimental.pallas import tpu_sc as plsc

D_MODEL = 768
LANES = 16
VECS_PER_ROW = D_MODEL // LANES
NUM_WORKERS = 32
CHUNK = 32
NBUF = 4


@functools.partial(jax.jit, static_argnames=("batch", "seq"))
def _emb_lookup_add(idx, token_table, pos_table, batch, seq):
    n = batch * seq
    pos_per_w = seq // NUM_WORKERS
    per_w = pos_per_w * batch
    n_halves = pos_per_w // CHUNK
    n_chunks = n_halves * batch
    mesh = plsc.VectorSubcoreMesh(core_axis_name="c", subcore_axis_name="s")

    @functools.partial(
        pl.kernel,
        mesh=mesh,
        out_type=jax.ShapeDtypeStruct((n, D_MODEL), jnp.float32),
        scratch_types=[
            pltpu.VMEM((per_w,), jnp.int32),
            pltpu.VMEM((CHUNK, D_MODEL), jnp.float32),
        ]
        + [pltpu.VMEM((CHUNK, D_MODEL), jnp.float32) for _ in range(NBUF)]
        + [pltpu.SemaphoreType.DMA for _ in range(2 * NBUF + 2)],
    )
    def k(idx_hbm, tok_hbm, pos_hbm, out_hbm, idx_v, posb, *bufs_sems):
        tokb = bufs_sems[:NBUF]
        gsem = bufs_sems[NBUF : 2 * NBUF]
        ssem = bufs_sems[2 * NBUF : 3 * NBUF]
        isem = bufs_sems[3 * NBUF]
        psem = bufs_sems[3 * NBUF + 1]

        wid = lax.axis_index("s") * 2 + lax.axis_index("c")
        pstart = wid * pos_per_w

        idx_cps = [
            pltpu.async_copy(
                idx_hbm.at[b, pl.ds(pstart, pos_per_w)],
                idx_v.at[pl.ds(b * pos_per_w, pos_per_w)],
                isem,
            )
            for b in range(batch)
        ]
        for cp in idx_cps:
            cp.wait()

        def chunk_gb(ck):
            return divmod(ck, batch)

        def start_gather(ck):
            g, b = chunk_gb(ck)
            return pltpu.async_copy(
                tok_hbm.at[idx_v.at[pl.ds(b * pos_per_w + g * CHUNK, CHUNK)]],
                tokb[ck % NBUF],
                gsem[ck % NBUF],
            )

        def out_row(ck):
            g, b = chunk_gb(ck)
            return b * seq + pstart + g * CHUNK

        gather_cps = {c: start_gather(c) for c in range(NBUF - 1)}
        store_cps = {}
        pos_cp = pltpu.async_copy(pos_hbm.at[pl.ds(pstart, CHUNK)], posb, psem)
        pos_cp.wait()

        for ck in range(n_chunks):
            p = ck % NBUF
            g, b = chunk_gb(ck)
            if b == 0 and g > 0:
                pltpu.sync_copy(pos_hbm.at[pl.ds(pstart + g * CHUNK, CHUNK)], posb)
            gather_cps[ck].wait()
            nk = ck + NBUF - 1
            if nk < n_chunks:
                if nk - NBUF >= 0:
                    store_cps[nk - NBUF].wait()
                gather_cps[nk] = start_gather(nk)

            buf = tokb[p]

            def row_body(r, _, buf=buf):
                for j in range(VECS_PER_ROW):
                    sl = pl.ds(j * LANES, LANES)
                    buf[r, sl] = buf[r, sl] + posb[r, sl]
                return 0

            lax.fori_loop(0, CHUNK, row_body, 0)
            store_cps[ck] = pltpu.async_copy(
                buf, out_hbm.at[pl.ds(out_row(ck), CHUNK)], ssem[p]
            )

        for ck in range(n_chunks - NBUF, n_chunks):
            store_cps[ck].wait()

    return k(idx, token_table, pos_table)


def kernel(inputs, token_table, pos_table):
    batch, seq = inputs.shape
    out = _emb_lookup_add(
        inputs.astype(jnp.int32), token_table, pos_table, batch, seq
    )
    return out.reshape(batch, seq, token_table.shape[1])
